# baseline (speedup 1.0000x reference)
import jax
import jax.numpy as jnp
from jax import lax
from jax.experimental import pallas as pl
from jax.experimental.pallas import tpu as pltpu
from jax.experimental.pallas import tpu_sc as plsc

_BATCH = 16384
_LATENT = 16
_EPR = 128 // _LATENT
_NC = 2
_NS = 16
_NW = _NC * _NS
_BPW = _BATCH // _NW
_HALF = _BPW // 2


def _mf_body(user_hbm, item_hbm, utab_hbm, itab_hbm, out_hbm,
             uidx_v, iidx_v, udiv, idiv, urows, irows, outv, usem, isem):
    wid = lax.axis_index("s") * _NC + lax.axis_index("c")
    base = wid * _BPW

    pltpu.sync_copy(user_hbm.at[pl.ds(base, _BPW)], uidx_v)
    pltpu.sync_copy(item_hbm.at[pl.ds(base, _BPW)], iidx_v)

    lane = lax.iota(jnp.int32, 16)

    for p in range(2):
        off = p * _HALF

        def split(g, carry):
            sl = pl.ds(g * 16, 16)
            gsl = pl.ds(off + g * 16, 16)
            udiv[sl] = uidx_v[gsl] >> 3
            idiv[sl] = iidx_v[gsl] >> 3
            return carry

        lax.fori_loop(0, _HALF // 16, split, 0)

        ucp = pltpu.async_copy(utab_hbm.at[udiv], urows, usem)
        icp = pltpu.async_copy(itab_hbm.at[idiv], irows, isem)
        ucp.wait()
        icp.wait()

        def group(g, carry):
            sl = pl.ds(g * 16, 16)
            gsl = pl.ds(off + g * 16, 16)
            rows = g * 16 + lane
            ubase = (uidx_v[gsl] & 7) << 4
            ibase = (iidx_v[gsl] & 7) << 4
            acc = (plsc.load_gather(urows, [rows, ubase])
                   * plsc.load_gather(irows, [rows, ibase]))
            for d in range(1, _LATENT):
                acc = acc + (plsc.load_gather(urows, [rows, ubase + d])
                             * plsc.load_gather(irows, [rows, ibase + d]))
            outv[gsl] = acc
            return carry

        lax.fori_loop(0, _HALF // 16, group, 0)

    pltpu.sync_copy(outv, out_hbm.at[pl.ds(base, _BPW)])


def kernel(user, item, user_table, item_table):
    user = user.astype(jnp.int32)
    item = item.astype(jnp.int32)
    utab = user_table.reshape(-1, 8 * _LATENT)
    itab = item_table.reshape(-1, 8 * _LATENT)
    mesh = plsc.VectorSubcoreMesh(core_axis_name="c", subcore_axis_name="s")
    f = pl.kernel(
        _mf_body,
        out_type=jax.ShapeDtypeStruct((_BATCH,), jnp.float32),
        mesh=mesh,
        compiler_params=pltpu.CompilerParams(needs_layout_passes=False),
        scratch_types=[
            pltpu.VMEM((_BPW,), jnp.int32),
            pltpu.VMEM((_BPW,), jnp.int32),
            pltpu.VMEM((_HALF,), jnp.int32),
            pltpu.VMEM((_HALF,), jnp.int32),
            pltpu.VMEM((_HALF, 8 * _LATENT), jnp.float32),
            pltpu.VMEM((_HALF, 8 * _LATENT), jnp.float32),
            pltpu.VMEM((_BPW,), jnp.float32),
            pltpu.SemaphoreType.DMA,
            pltpu.SemaphoreType.DMA,
        ],
    )
    return f(user, item, utab, itab)


# speedup vs baseline: 2.7246x; 2.7246x over previous
import jax
import jax.numpy as jnp
from jax import lax
from jax.experimental import pallas as pl
from jax.experimental.pallas import tpu as pltpu
from jax.experimental.pallas import tpu_sc as plsc

_BATCH = 16384
_LATENT = 16
_NC = 2
_NS = 16
_NW = _NC * _NS
_BPW = _BATCH // _NW
_HALF = _BPW // 2

_NUSERS = 1000000
_NTILE = (_NUSERS + 127) // 128
_NROWS = _NTILE * 16
_STEPS = 2 * ((_NTILE + 2 * _NW - 1) // (2 * _NW))
_NTT = _STEPS // 2


def _relayout_body(utab, itab, urm, irm,
                   tin_u0, tin_u1, tin_i0, tin_i1,
                   tout_u0, tout_u1, tout_i0, tout_i1,
                   siu0, siu1, sii0, sii1, sou0, sou1, soi0, soi1):
    wid = lax.axis_index("s") * _NC + lax.axis_index("c")
    tin = ((tin_u0, tin_i0), (tin_u1, tin_i1))
    tout = ((tout_u0, tout_i0), (tout_u1, tout_i1))
    sin = ((siu0, sii0), (siu1, sii1))
    sout = ((sou0, soi0), (sou1, soi1))
    tabs = (utab, itab)
    outs = (urm, irm)

    lane = lax.iota(jnp.int32, 16)
    r0 = lane >> 3
    c0 = (lane & 7) << 4

    def col(t):
        return jnp.minimum(wid + _NW * t, _NTILE - 1)

    def fetch(t, s):
        j = col(t)
        src_off = pl.multiple_of(j * 128, 128)
        for k in range(2):
            pltpu.async_copy(tabs[k].at[:, pl.ds(src_off, 128)],
                             tin[s][k], sin[s][k])

    def wait_fetch(s):
        for k in range(2):
            pltpu.make_async_copy(tabs[k].at[:, pl.ds(0, 128)],
                                  tin[s][k], sin[s][k]).wait()

    def store(t, s):
        j = col(t)
        dst_off = pl.multiple_of(j * 16, 16)
        for k in range(2):
            pltpu.async_copy(tout[s][k],
                             outs[k].at[pl.ds(dst_off, 16), :], sout[s][k])

    def wait_store(s):
        for k in range(2):
            pltpu.make_async_copy(tout[s][k],
                                  outs[k].at[pl.ds(0, 16), :],
                                  sout[s][k]).wait()

    fetch(0, 0)
    fetch(1, 1)

    def step(tt, carry):
        for s in range(2):
            t = 2 * tt + s
            wait_fetch(s)

            @pl.when(tt >= 1)
            def _():
                wait_store(s)

            for k in range(2):
                src = tin[s][k]
                dst = tout[s][k]
                for g in range(8):
                    rv = r0 + 2 * g
                    for d in range(_LATENT):
                        v = src[d, pl.ds(g * 16, 16)]
                        plsc.store_scatter(dst, [rv, c0 + d], v)
            store(t, s)
            fetch(t + 2, s)
        return carry

    lax.fori_loop(0, _NTT, step, 0)

    for s in range(2):
        wait_fetch(s)
        wait_store(s)


def _mf_body(user_hbm, item_hbm, utab_hbm, itab_hbm, out_hbm,
             uidx_v, iidx_v, udiv, idiv, urows, irows, outv, usem, isem):
    wid = lax.axis_index("s") * _NC + lax.axis_index("c")
    base = wid * _BPW

    pltpu.sync_copy(user_hbm.at[pl.ds(base, _BPW)], uidx_v)
    pltpu.sync_copy(item_hbm.at[pl.ds(base, _BPW)], iidx_v)

    lane = lax.iota(jnp.int32, 16)

    for p in range(2):
        off = p * _HALF

        def split(g, carry):
            sl = pl.ds(g * 16, 16)
            gsl = pl.ds(off + g * 16, 16)
            udiv[sl] = uidx_v[gsl] >> 3
            idiv[sl] = iidx_v[gsl] >> 3
            return carry

        lax.fori_loop(0, _HALF // 16, split, 0)

        ucp = pltpu.async_copy(utab_hbm.at[udiv], urows, usem)
        icp = pltpu.async_copy(itab_hbm.at[idiv], irows, isem)
        ucp.wait()
        icp.wait()

        def group(g, carry):
            gsl = pl.ds(off + g * 16, 16)
            rows = g * 16 + lane
            ubase = (uidx_v[gsl] & 7) << 4
            ibase = (iidx_v[gsl] & 7) << 4
            acc = (plsc.load_gather(urows, [rows, ubase])
                   * plsc.load_gather(irows, [rows, ibase]))
            for d in range(1, _LATENT):
                acc = acc + (plsc.load_gather(urows, [rows, ubase + d])
                             * plsc.load_gather(irows, [rows, ibase + d]))
            outv[gsl] = acc
            return carry

        lax.fori_loop(0, _HALF // 16, group, 0)

    pltpu.sync_copy(outv, out_hbm.at[pl.ds(base, _BPW)])


def kernel(user, item, user_table, item_table):
    user = user.astype(jnp.int32)
    item = item.astype(jnp.int32)
    mesh = plsc.VectorSubcoreMesh(core_axis_name="c", subcore_axis_name="s")

    relayout = pl.kernel(
        _relayout_body,
        out_type=(jax.ShapeDtypeStruct((_NROWS, 128), jnp.float32),
                  jax.ShapeDtypeStruct((_NROWS, 128), jnp.float32)),
        mesh=mesh,
        compiler_params=pltpu.CompilerParams(needs_layout_passes=False),
        scratch_types=(
            [pltpu.VMEM((_LATENT, 128), jnp.float32) for _ in range(8)]
            + [pltpu.SemaphoreType.DMA for _ in range(8)]
        ),
    )
    urm, irm = relayout(user_table.T, item_table.T)

    gather_dot = pl.kernel(
        _mf_body,
        out_type=jax.ShapeDtypeStruct((_BATCH,), jnp.float32),
        mesh=mesh,
        compiler_params=pltpu.CompilerParams(needs_layout_passes=False),
        scratch_types=[
            pltpu.VMEM((_BPW,), jnp.int32),
            pltpu.VMEM((_BPW,), jnp.int32),
            pltpu.VMEM((_HALF,), jnp.int32),
            pltpu.VMEM((_HALF,), jnp.int32),
            pltpu.VMEM((_HALF, 128), jnp.float32),
            pltpu.VMEM((_HALF, 128), jnp.float32),
            pltpu.VMEM((_BPW,), jnp.float32),
            pltpu.SemaphoreType.DMA,
            pltpu.SemaphoreType.DMA,
        ],
    )
    return gather_dot(user, item, urm, irm)


# speedup vs baseline: 6.3387x; 2.3265x over previous
import jax
import jax.numpy as jnp
from jax import lax
from jax.experimental import pallas as pl
from jax.experimental.pallas import tpu as pltpu
from jax.experimental.pallas import tpu_sc as plsc

_BATCH = 16384
_LATENT = 16
_NC = 2
_NS = 16
_NW = _NC * _NS
_BPW = _BATCH // _NW
_NBUF = 16
_NG = _BPW // _NBUF


def _mf_body(user_hbm, item_hbm, utab, itab, out_hbm,
             uidx, iidx, outs, pbuf, *rest):
    tu = rest[0:_NBUF]
    ti = rest[_NBUF:2 * _NBUF]
    sm = rest[2 * _NBUF:3 * _NBUF]

    wid = lax.axis_index("s") * _NC + lax.axis_index("c")
    base = wid * _BPW

    pltpu.sync_copy(user_hbm.at[pl.ds(base, _BPW)], uidx)
    pltpu.sync_copy(item_hbm.at[pl.ds(base, _BPW)], iidx)

    lane = lax.iota(jnp.int32, 16)
    zv = jnp.zeros((16,), jnp.int32)

    def fetch(uv, iv, s):
        uoff = pl.multiple_of((uv[s] >> 7) * 128, 128)
        ioff = pl.multiple_of((iv[s] >> 7) * 128, 128)
        pltpu.async_copy(utab.at[:, pl.ds(uoff, 128)], tu[s], sm[s])
        pltpu.async_copy(itab.at[:, pl.ds(ioff, 128)], ti[s], sm[s])

    uv0 = uidx[pl.ds(0, 16)]
    iv0 = iidx[pl.ds(0, 16)]
    for s in range(_NBUF):
        fetch(uv0, iv0, s)

    def step(g, carry):
        sl = pl.ds(g * 16, 16)
        uv = uidx[sl]
        iv = iidx[sl]
        gn = jnp.minimum(g + 1, _NG - 1)
        nsl = pl.ds(pl.multiple_of(gn * 16, 16), 16)
        uvn = uidx[nsl]
        ivn = iidx[nsl]
        for s in range(_NBUF):
            pltpu.make_async_copy(utab.at[:, pl.ds(0, 128)],
                                  tu[s], sm[s]).wait()
            pltpu.make_async_copy(itab.at[:, pl.ds(0, 128)],
                                  ti[s], sm[s]).wait()
            cu = zv + (uv[s] & 127)
            ci = zv + (iv[s] & 127)
            ue = plsc.load_gather(tu[s], [lane, cu])
            ie = plsc.load_gather(ti[s], [lane, ci])
            pbuf[s] = ue * ie
            fetch(uvn, ivn, s)
        acc = plsc.load_gather(pbuf, [lane, zv])
        for d in range(1, _LATENT):
            acc = acc + plsc.load_gather(pbuf, [lane, zv + d])
        outs[sl] = acc
        return carry

    lax.fori_loop(0, _NG, step, 0)

    for s in range(_NBUF):
        pltpu.make_async_copy(utab.at[:, pl.ds(0, 128)],
                              tu[s], sm[s]).wait()
        pltpu.make_async_copy(itab.at[:, pl.ds(0, 128)],
                              ti[s], sm[s]).wait()

    pltpu.sync_copy(outs, out_hbm.at[pl.ds(base, _BPW)])


def kernel(user, item, user_table, item_table):
    user = user.astype(jnp.int32)
    item = item.astype(jnp.int32)
    mesh = plsc.VectorSubcoreMesh(core_axis_name="c", subcore_axis_name="s")
    f = pl.kernel(
        _mf_body,
        out_type=jax.ShapeDtypeStruct((_BATCH,), jnp.float32),
        mesh=mesh,
        compiler_params=pltpu.CompilerParams(needs_layout_passes=False),
        scratch_types=(
            [pltpu.VMEM((_BPW,), jnp.int32),
             pltpu.VMEM((_BPW,), jnp.int32),
             pltpu.VMEM((_BPW,), jnp.float32),
             pltpu.VMEM((_NBUF, _LATENT), jnp.float32)]
            + [pltpu.VMEM((_LATENT, 128), jnp.float32)
               for _ in range(2 * _NBUF)]
            + [pltpu.SemaphoreType.DMA for _ in range(_NBUF)]
        ),
    )
    return f(user, item, user_table.T, item_table.T)
